# baseline (speedup 1.0000x reference)
import functools
import math

import jax
import jax.numpy as jnp
from jax import lax
from jax.experimental import pallas as pl
from jax.experimental.pallas import tpu as pltpu
from jax.experimental.pallas import tpu_sc as plsc

_BASE_GSD = 10.0
_LN10K = math.log(10000.0)

_NC = 2
_NS = 16
_NW = _NC * _NS
_LANES = 16


def _fiota(shape, dim):
    return jax.lax.broadcasted_iota(jnp.int32, shape, dim).astype(jnp.float32)


def _tables_body(months_ref, ch_ref, res_ref, a_ref, sp_ref, *, B, H, W, T, C, D):
    dq = D // 4
    f32 = jnp.float32
    res = res_ref[0]

    k16 = _fiota((T, dq // 2), 1)
    om16 = jnp.exp(k16 * (-_LN10K / (dq // 2)))
    ang_t = _fiota((T, dq // 2), 0) * om16
    pos_tab = jnp.concatenate([jnp.sin(ang_t), jnp.cos(ang_t)], axis=-1)

    mth = months_ref[...].astype(f32)
    ang_m = jnp.broadcast_to(mth * (2.0 * math.pi / 12.0), (B, T, dq // 2))
    m_emb = jnp.concatenate([jnp.sin(ang_m), jnp.cos(ang_m)], axis=-1)

    ch = ch_ref[...]
    a_ref[...] = jnp.concatenate(
        [
            jnp.broadcast_to(ch[None, None], (B, T, C, dq)),
            jnp.broadcast_to(pos_tab[None, :, None], (B, T, C, dq)),
            jnp.broadcast_to(m_emb[:, :, None], (B, T, C, dq)),
            jnp.zeros((B, T, C, dq), f32),
        ],
        axis=-1,
    ).reshape(B, T * C, D)

    k8 = _fiota((W, dq // 4), 1)
    om8 = jnp.exp(k8 * (-_LN10K / (dq // 4)))
    ang_w = _fiota((W, dq // 4), 0) * res * om8
    emb_w = jnp.concatenate([jnp.sin(ang_w), jnp.cos(ang_w)], axis=-1)
    k8h = _fiota((H, dq // 4), 1)
    om8h = jnp.exp(k8h * (-_LN10K / (dq // 4)))
    ang_h = _fiota((H, dq // 4), 0) * res * om8h
    emb_h = jnp.concatenate([jnp.sin(ang_h), jnp.cos(ang_h)], axis=-1)
    sp_ref[...] = jnp.concatenate(
        [
            jnp.zeros((H, W, 3 * dq), f32),
            jnp.broadcast_to(emb_w[None], (H, W, dq // 2)),
            jnp.broadcast_to(emb_h[:, None], (H, W, dq // 2)),
        ],
        axis=-1,
    ).reshape(H * W, D)


def _tables_tc(months3, channel_emb_s2, res, B, H, W, T, C, D):
    dq = D // 4
    body = functools.partial(_tables_body, B=B, H=H, W=W, T=T, C=C, D=D)
    return pl.pallas_call(
        body,
        in_specs=[
            pl.BlockSpec((B, T, 1), lambda: (0, 0, 0)),
            pl.BlockSpec((C, dq), lambda: (0, 0)),
            pl.BlockSpec(memory_space=pltpu.SMEM),
        ],
        out_specs=[
            pl.BlockSpec((B, T * C, D), lambda: (0, 0, 0)),
            pl.BlockSpec((H * W, D), lambda: (0, 0)),
        ],
        out_shape=[
            jax.ShapeDtypeStruct((B, T * C, D), jnp.float32),
            jax.ShapeDtypeStruct((H * W, D), jnp.float32),
        ],
    )(months3, channel_emb_s2, res)


def _sc_add(s2f, a_tab, sp_tab, P, R, D, HW):
    ppw = P // _NW
    nq = (3 * D) // (4 * _LANES)
    runroll = 4
    RD = R * D
    mesh = plsc.VectorSubcoreMesh(core_axis_name="c", subcore_axis_name="s")

    @functools.partial(
        pl.kernel,
        mesh=mesh,
        out_type=jax.ShapeDtypeStruct((P * RD,), jnp.float32),
        scratch_types=[
            pltpu.VMEM((RD,), jnp.float32),
            pltpu.VMEM((ppw * D,), jnp.float32),
            pltpu.VMEM((RD,), jnp.float32),
            pltpu.VMEM((RD,), jnp.float32),
            pltpu.VMEM((RD,), jnp.float32),
            pltpu.VMEM((RD,), jnp.float32),
            pltpu.SemaphoreType.DMA,
            pltpu.SemaphoreType.DMA,
            pltpu.SemaphoreType.DMA,
            pltpu.SemaphoreType.DMA,
        ],
    )
    def k(s2_hbm, a_hbm, sp_hbm, out_hbm, a_v, sp_v, bi0, bi1, bo0, bo1,
          si0, si1, so0, so1):
        wid = lax.axis_index("s") * _NC + lax.axis_index("c")
        base = wid * ppw
        b = base // HW
        pltpu.sync_copy(a_hbm.at[pl.ds(b * RD, RD)], a_v)
        pltpu.sync_copy(sp_hbm.at[pl.ds((base - b * HW) * D, ppw * D)], sp_v)

        bi = [bi0, bi1]
        bo = [bo0, bo1]
        sin_ = [si0, si1]
        sout = [so0, so1]
        in_desc = [
            pltpu.async_copy(s2_hbm.at[pl.ds((base + 0) * RD, RD)], bi[0], sin_[0]),
            pltpu.async_copy(s2_hbm.at[pl.ds((base + 1) * RD, RD)], bi[1], sin_[1]),
        ]
        out_desc = [None, None]
        for p in range(ppw):
            j = p % 2
            in_desc[j].wait()
            if p >= 2:
                out_desc[j].wait()
            s_lo = sp_v[pl.ds(p * D + nq * _LANES, _LANES)]
            s_hi = sp_v[pl.ds(p * D + (nq + 1) * _LANES, _LANES)]

            def row4(r4, c2, _bi=bi[j], _bo=bo[j], _lo=s_lo, _hi=s_hi):
                for kk in range(runroll):
                    r = r4 * runroll + kk
                    for q in range(nq):
                        sl = pl.ds(r * D + _LANES * q, _LANES)
                        _bo[sl] = _bi[sl] + a_v[sl]
                    sl6 = pl.ds(r * D + nq * _LANES, _LANES)
                    sl7 = pl.ds(r * D + (nq + 1) * _LANES, _LANES)
                    _bo[sl6] = _bi[sl6] + _lo
                    _bo[sl7] = _bi[sl7] + _hi
                return c2

            lax.fori_loop(0, R // runroll, row4, 0)
            out_desc[j] = pltpu.async_copy(
                bo[j], out_hbm.at[pl.ds((base + p) * RD, RD)], sout[j])
            if p + 2 < ppw:
                in_desc[j] = pltpu.async_copy(
                    s2_hbm.at[pl.ds((base + p + 2) * RD, RD)], bi[j], sin_[j])
        out_desc[0].wait()
        out_desc[1].wait()

    return k(s2f, a_tab, sp_tab)


def kernel(s2, months, patch_size, input_res, channel_emb_s2):
    b, h, w, t, c_g, d = s2.shape
    res = (jnp.asarray(input_res, jnp.float32) * patch_size / _BASE_GSD).reshape(1)
    months3 = months.reshape(b, t, 1)
    a_tab, sp_tab = _tables_tc(months3, channel_emb_s2, res, b, h, w, t, c_g, d)
    out = _sc_add(
        s2.reshape(-1), a_tab.reshape(-1), sp_tab.reshape(-1),
        b * h * w, t * c_g, d, h * w)
    return out.reshape(s2.shape)


# speedup vs baseline: 1.9340x; 1.9340x over previous
import functools
import math

import jax
import jax.numpy as jnp
from jax.experimental import pallas as pl
from jax.experimental.pallas import tpu as pltpu

_BASE_GSD = 10.0
_LN10K = math.log(10000.0)


def _body(s2_ref, months_ref, ch_ref, res_ref, out_ref, *, H, W, T, C, D, RH):
    dq = D // 4
    f32 = jnp.float32
    i = pl.program_id(0)
    res = res_ref[0]

    def fiota(shape, dim):
        return jax.lax.broadcasted_iota(jnp.int32, shape, dim).astype(f32)

    k16 = fiota((T, dq // 2), 1)
    om16 = jnp.exp(k16 * (-_LN10K / (dq // 2)))
    tv = fiota((T, dq // 2), 0)
    ang_t = tv * om16
    pos_tab = jnp.concatenate([jnp.sin(ang_t), jnp.cos(ang_t)], axis=-1)

    mth = months_ref[0].astype(f32)
    ang_m = jnp.broadcast_to(mth * (2.0 * math.pi / 12.0), (T, dq // 2))
    m_emb = jnp.concatenate([jnp.sin(ang_m), jnp.cos(ang_m)], axis=-1)

    ch = ch_ref[...]
    base = jnp.concatenate(
        [
            jnp.broadcast_to(ch[None], (T, C, dq)),
            jnp.broadcast_to(pos_tab[:, None], (T, C, dq)),
            jnp.broadcast_to(m_emb[:, None], (T, C, dq)),
            jnp.zeros((T, C, dq), f32),
        ],
        axis=-1,
    )

    k8 = fiota((W, dq // 4), 1)
    om8 = jnp.exp(k8 * (-_LN10K / (dq // 4)))
    jv = fiota((W, dq // 4), 0) * res
    ang_w = jv * om8
    emb_w = jnp.concatenate([jnp.sin(ang_w), jnp.cos(ang_w)], axis=-1)
    hbase = (i * RH) % H
    k8r = fiota((RH, dq // 4), 1)
    om8r = jnp.exp(k8r * (-_LN10K / (dq // 4)))
    hv = (fiota((RH, dq // 4), 0) + hbase.astype(f32)) * res
    ang_h = hv * om8r
    emb_h = jnp.concatenate([jnp.sin(ang_h), jnp.cos(ang_h)], axis=-1)
    sp = jnp.concatenate(
        [
            jnp.broadcast_to(emb_w[None], (RH, W, dq // 2)),
            jnp.broadcast_to(emb_h[:, None], (RH, W, dq // 2)),
        ],
        axis=-1,
    )
    spfull = jnp.concatenate([jnp.zeros((RH, W, 3 * dq), f32), sp], axis=-1)

    out_ref[0] = (
        s2_ref[0]
        + base[None, None]
        + spfull[:, :, None, None, :]
    )


def kernel(s2, months, patch_size, input_res, channel_emb_s2):
    b, h, w, t, c_g, d = s2.shape
    RH = 4
    res = (jnp.asarray(input_res, jnp.float32) * patch_size / _BASE_GSD).reshape(1)
    months3 = months.reshape(b, t, 1)
    steps_per_b = h // RH

    body = functools.partial(_body, H=h, W=w, T=t, C=c_g, D=d, RH=RH)
    out = pl.pallas_call(
        body,
        grid=(b * h // RH,),
        in_specs=[
            pl.BlockSpec(
                (1, RH, w, t, c_g, d),
                lambda i, _s=steps_per_b: (i // _s, i % _s, 0, 0, 0, 0)),
            pl.BlockSpec((1, t, 1), lambda i, _s=steps_per_b: (i // _s, 0, 0)),
            pl.BlockSpec((c_g, d // 4), lambda i: (0, 0)),
            pl.BlockSpec(memory_space=pltpu.SMEM),
        ],
        out_specs=pl.BlockSpec(
            (1, RH, w, t, c_g, d),
            lambda i, _s=steps_per_b: (i // _s, i % _s, 0, 0, 0, 0)),
        out_shape=jax.ShapeDtypeStruct(s2.shape, s2.dtype),
    )(s2, months3, channel_emb_s2, res)
    return out


# speedup vs baseline: 2.5564x; 1.3218x over previous
import functools
import math

import jax
import jax.numpy as jnp
from jax.experimental import pallas as pl
from jax.experimental.pallas import tpu as pltpu

_BASE_GSD = 10.0
_LN10K = math.log(10000.0)


def _body(s2_ref, months_ref, ch_ref, res_ref, out_ref, *, H, W, T, C, D, RH):
    dq = D // 4
    f32 = jnp.float32
    i = pl.program_id(0)
    res = res_ref[0]

    def fiota(shape, dim):
        return jax.lax.broadcasted_iota(jnp.int32, shape, dim).astype(f32)

    k16 = fiota((T, dq // 2), 1)
    om16 = jnp.exp(k16 * (-_LN10K / (dq // 2)))
    tv = fiota((T, dq // 2), 0)
    ang_t = tv * om16
    pos_tab = jnp.concatenate([jnp.sin(ang_t), jnp.cos(ang_t)], axis=-1)

    mth = months_ref[0].astype(f32)
    ang_m = jnp.broadcast_to(mth * (2.0 * math.pi / 12.0), (T, dq // 2))
    m_emb = jnp.concatenate([jnp.sin(ang_m), jnp.cos(ang_m)], axis=-1)

    ch = ch_ref[...]
    base = jnp.concatenate(
        [
            jnp.broadcast_to(ch[None], (T, C, dq)).reshape(T * C, dq),
            jnp.broadcast_to(pos_tab[:, None], (T, C, dq)).reshape(T * C, dq),
            jnp.broadcast_to(m_emb[:, None], (T, C, dq)).reshape(T * C, dq),
            jnp.zeros((T * C, dq), f32),
        ],
        axis=-1,
    )

    k8 = fiota((W, dq // 4), 1)
    om8 = jnp.exp(k8 * (-_LN10K / (dq // 4)))
    jv = fiota((W, dq // 4), 0) * res
    ang_w = jv * om8
    emb_w = jnp.concatenate([jnp.sin(ang_w), jnp.cos(ang_w)], axis=-1)
    hbase = (i * RH) % H
    k8r = fiota((RH, dq // 4), 1)
    om8r = jnp.exp(k8r * (-_LN10K / (dq // 4)))
    hv = (fiota((RH, dq // 4), 0) + hbase.astype(f32)) * res
    ang_h = hv * om8r
    emb_h = jnp.concatenate([jnp.sin(ang_h), jnp.cos(ang_h)], axis=-1)
    sp = jnp.concatenate(
        [
            jnp.broadcast_to(emb_w[None], (RH, W, dq // 2)),
            jnp.broadcast_to(emb_h[:, None], (RH, W, dq // 2)),
        ],
        axis=-1,
    )
    spfull = jnp.concatenate([jnp.zeros((RH, W, 3 * dq), f32), sp], axis=-1)

    out_ref[...] = s2_ref[...] + base[None, None] + spfull[:, :, None, :]


def kernel(s2, months, patch_size, input_res, channel_emb_s2):
    b, h, w, t, c_g, d = s2.shape
    RH = 8
    res = (jnp.asarray(input_res, jnp.float32) * patch_size / _BASE_GSD).reshape(1)
    s2v = s2.reshape(b * h, w, t * c_g, d)
    months3 = months.reshape(b, t, 1)
    steps_per_b = h // RH

    body = functools.partial(_body, H=h, W=w, T=t, C=c_g, D=d, RH=RH)
    out = pl.pallas_call(
        body,
        grid=(b * h // RH,),
        in_specs=[
            pl.BlockSpec((RH, w, t * c_g, d), lambda i: (i, 0, 0, 0)),
            pl.BlockSpec((1, t, 1), lambda i, _s=steps_per_b: (i // _s, 0, 0)),
            pl.BlockSpec((c_g, d // 4), lambda i: (0, 0)),
            pl.BlockSpec(memory_space=pltpu.SMEM),
        ],
        out_specs=pl.BlockSpec((RH, w, t * c_g, d), lambda i: (i, 0, 0, 0)),
        compiler_params=pltpu.CompilerParams(vmem_limit_bytes=128 * 1024 * 1024),
        out_shape=jax.ShapeDtypeStruct(s2v.shape, s2.dtype),
    )(s2v, months3, channel_emb_s2, res)
    return out.reshape(s2.shape)
